# baseline (speedup 1.0000x reference)
import functools

import jax
import jax.numpy as jnp
from jax import lax
from jax.experimental import pallas as pl
from jax.experimental.pallas import tpu as pltpu
from jax.experimental.pallas import tpu_sc as plsc

N = 10000
E = 320000
CHUNK = 128
NSC = 2
NTILES = 16
NW = NSC * NTILES
EPT = E // NW
B = 80
NB = EPT // B
NSEG = 5
NBS = NB // NSEG
NP = 10240
ROWS_PT = NP // NTILES
EPS = 1e-5
_HEAD_NAMES = ['port_surge', 'rail_congestion', 'terminal_utilization',
               'drayage_delay', 'chokepoint_risk']

@functools.cache
def _mesh():
    return plsc.VectorSubcoreMesh(core_axis_name="c", subcore_axis_name="s",
                                  num_cores=NSC, num_subcores=NTILES)


def _make_sc_agg(nc, with_deg):
    npass = nc + (1 if with_deg else 0)
    out_type = jax.ShapeDtypeStruct((NSC * npass * NP, CHUNK), jnp.float32)
    scratch = [
        pltpu.VMEM_SHARED((NP, CHUNK), jnp.float32),
        pltpu.VMEM((NBS, B), jnp.int32),
        pltpu.VMEM((NBS, B), jnp.int32),
        pltpu.VMEM((NBS, B), jnp.int32),
        pltpu.VMEM((NBS, B), jnp.int32),
        pltpu.VMEM((B, CHUNK), jnp.float32),
        pltpu.VMEM((B, CHUNK), jnp.float32),
        pltpu.SemaphoreType.DMA,
        pltpu.SemaphoreType.DMA,
        pltpu.SemaphoreType.DMA,
        pltpu.SemaphoreType.DMA,
        pltpu.SemaphoreType.DMA,
    ]
    if with_deg:
        scratch.append(pltpu.VMEM((B, CHUNK), jnp.float32))

    def body(h_hbm, srcoff_hbm, dst_hbm, zeros_hbm, parts_hbm, acc,
             src_v0, src_v1, dst_v0, dst_v1, msgs0, msgs1,
             sem0, sem1, ssem0, ssem1, isem, *rest):
        ones_v = rest[0] if with_deg else None
        c = lax.axis_index("c")
        s = lax.axis_index("s")
        wid = c * NTILES + s
        row0 = s * ROWS_PT

        if with_deg:
            ov = jnp.ones((16,), jnp.float32)

            def fill_ones(r, carry):
                for k in range(CHUNK // 16):
                    ones_v[r, k * 16:(k + 1) * 16] = ov
                return carry
            lax.fori_loop(0, B, fill_ones, 0)

        for cc in range(npass):
            pltpu.sync_copy(zeros_hbm, acc.at[pl.ds(row0, ROWS_PT)])
            plsc.subcore_barrier()

            if cc < nc:
                sbufs = (src_v0, src_v1)
                dbufs = (dst_v0, dst_v1)

                def iload(seg, issue):
                    sv, dv = sbufs[seg % 2], dbufs[seg % 2]
                    cps = [(dst_hbm.at[wid * NSEG + seg], dv),
                           (srcoff_hbm.at[(cc * NW + wid) * NSEG + seg], sv)]
                    for sref, dref in cps:
                        if issue:
                            pltpu.async_copy(sref, dref, isem)
                        else:
                            pltpu.make_async_copy(sref, dref, isem).wait()

                iload(0, True)
                for seg in range(NSEG):
                    src_v, dst_v = sbufs[seg % 2], dbufs[seg % 2]
                    iload(seg, False)
                    if seg + 1 < NSEG:
                        iload(seg + 1, True)

                    def gat(j, buf, sem):
                        pltpu.async_copy(h_hbm.at[src_v.at[j]], buf, sem)

                    def gwait(j, buf, sem):
                        pltpu.make_async_copy(h_hbm.at[src_v.at[j]], buf, sem
                                              ).wait()

                    def sstart(j, buf, sem):
                        pltpu.async_copy(buf, acc.at[dst_v.at[j]], sem,
                                         add=True)

                    def swait(j, buf, sem):
                        pltpu.make_async_copy(buf, acc.at[dst_v.at[j]], sem
                                              ).wait()

                    gat(0, msgs0, sem0)
                    gwait(0, msgs0, sem0)
                    sstart(0, msgs0, ssem0)
                    gat(1, msgs1, sem1)

                    def bstep(k, carry):
                        j = 2 * k + 1
                        gwait(j, msgs1, sem1)
                        sstart(j, msgs1, ssem1)
                        swait(j - 1, msgs0, ssem0)
                        gat(j + 1, msgs0, sem0)
                        gwait(j + 1, msgs0, sem0)
                        sstart(j + 1, msgs0, ssem0)
                        swait(j, msgs1, ssem1)
                        gat(j + 2, msgs1, sem1)
                        return carry
                    lax.fori_loop(0, (NBS - 3) // 2, bstep, 0)
                    j = NBS - 2
                    gwait(j, msgs1, sem1)
                    sstart(j, msgs1, ssem1)
                    swait(j - 1, msgs0, ssem0)
                    gat(j + 1, msgs0, sem0)
                    gwait(j + 1, msgs0, sem0)
                    sstart(j + 1, msgs0, ssem0)
                    swait(j, msgs1, ssem1)
                    swait(j + 1, msgs0, ssem0)
            else:
                for seg in range(NSEG):
                    pltpu.sync_copy(dst_hbm.at[wid * NSEG + seg], dst_v0)

                    def bstep(j, carry):
                        pltpu.sync_copy(ones_v, acc.at[dst_v0.at[j]], add=True)
                        return carry
                    lax.fori_loop(0, NBS, bstep, 0)

            plsc.subcore_barrier()
            pltpu.sync_copy(acc.at[pl.ds(row0, ROWS_PT)],
                            parts_hbm.at[pl.ds((c * npass + cc) * NP + row0, ROWS_PT)])
            if cc + 1 < npass:
                plsc.subcore_barrier()

    return pl.kernel(body, out_type=out_type, mesh=_mesh(),
                     scratch_types=tuple(scratch))


_R = 2000


def _pre_tc_body(nc, h_ref, wr_ref, b_ref, out_ref):
    h = jnp.concatenate([h_ref[i] for i in range(nc)], axis=-1)
    out_ref[...] = (jnp.dot(h, wr_ref[...], preferred_element_type=jnp.float32)
                    + b_ref[...])


def _make_pre_tc(nc):
    return pl.pallas_call(
        functools.partial(_pre_tc_body, nc),
        grid=(N // _R,),
        in_specs=[
            pl.BlockSpec((nc, _R, CHUNK), lambda i: (0, i, 0)),
            pl.BlockSpec((nc * CHUNK, 512), lambda i: (0, 0)),
            pl.BlockSpec((1, 512), lambda i: (0, 0)),
        ],
        out_specs=pl.BlockSpec((_R, 512), lambda i: (i, 0)),
        out_shape=jax.ShapeDtypeStruct((N, 512), jnp.float32),
    )


def _post_tc_body(nc, parts_ref, degp_ref, ypre_ref, wl_ref, out_ref):
    p = parts_ref[...]
    aggc = p[0] + p[1]
    agg = jnp.concatenate([aggc[i] for i in range(nc)], axis=-1)
    dp = degp_ref[...]
    deg = jnp.max(dp[0, 0] + dp[1, 0], axis=-1, keepdims=True)
    inv = 1.0 / jnp.maximum(deg, 1.0)
    y = jnp.dot(agg * inv, wl_ref[...], preferred_element_type=jnp.float32)
    y = jnp.maximum(y + ypre_ref[...], 0.0)
    for k in range(4):
        out_ref[k] = y[:, k * 128:(k + 1) * 128]


def _make_post_tc(nc):
    return pl.pallas_call(
        functools.partial(_post_tc_body, nc),
        grid=(N // _R,),
        in_specs=[
            pl.BlockSpec((NSC, nc, _R, CHUNK), lambda i: (0, 0, i, 0)),
            pl.BlockSpec((NSC, 1, _R, CHUNK), lambda i: (0, 1, i, 0)),
            pl.BlockSpec((_R, 512), lambda i: (i, 0)),
            pl.BlockSpec((nc * CHUNK, 512), lambda i: (0, 0)),
        ],
        out_specs=pl.BlockSpec((4, _R, CHUNK), lambda i: (0, i, 0)),
        out_shape=jax.ShapeDtypeStruct((4, N, CHUNK), jnp.float32),
    )


def _heads_tc_body(h_ref, w1_ref, b1_ref, w2_ref, b2_ref, out_ref):
    h = jnp.concatenate([h_ref[i] for i in range(4)], axis=-1)
    z = jnp.dot(h, w1_ref[...], preferred_element_type=jnp.float32)
    z = jnp.maximum(z + b1_ref[...], 0.0)
    o = jnp.dot(z, w2_ref[...], preferred_element_type=jnp.float32)
    out_ref[...] = jax.nn.sigmoid(o + b2_ref[...])


_heads_tc = pl.pallas_call(
    _heads_tc_body,
    grid=(N // _R,),
    in_specs=[
        pl.BlockSpec((4, _R, CHUNK), lambda i: (0, i, 0)),
        pl.BlockSpec((512, 1280), lambda i: (0, 0)),
        pl.BlockSpec((1, 1280), lambda i: (0, 0)),
        pl.BlockSpec((1280, 128), lambda i: (0, 0)),
        pl.BlockSpec((1, 128), lambda i: (0, 0)),
    ],
    out_specs=pl.BlockSpec((_R, 128), lambda i: (i, 0)),
    out_shape=jax.ShapeDtypeStruct((N, 128), jnp.float32),
)


def kernel(x, edge_index, params):
    src = edge_index[0]
    dst = edge_index[1]
    perm = jnp.argsort(dst)
    src = jnp.where(dst[perm] >= 0, src, 0)
    dst_r = dst.reshape(NW * NSEG, NBS, B)
    src_r = src.reshape(NW * NSEG, NBS, B)
    srcoff1 = src_r
    srcoff4 = (src_r[None] + (jnp.arange(4, dtype=jnp.int32) * N)[:, None, None, None]
               ).reshape(4 * NW * NSEG, NBS, B)
    zeros_rows = jnp.zeros((ROWS_PT, CHUNK), jnp.float32)

    wls, wrs, biases = [], [], []
    for lp in params['layers']:
        g = lp['gamma'] / jnp.sqrt(lp['rv'] + EPS)
        wls.append(lp['Wl'] * g[None, :])
        wrs.append(lp['Wr'] * g[None, :])
        biases.append(((lp['bl'] - lp['rm']) * g + lp['beta']).reshape(1, 512))

    sc1 = _make_sc_agg(1, True)
    sc4 = _make_sc_agg(4, False)
    pre1, post1 = _make_pre_tc(1), _make_post_tc(1)
    pre4, post4 = _make_pre_tc(4), _make_post_tc(4)

    parts1 = sc1(x, srcoff1, dst_r, zeros_rows).reshape(NSC, 2, NP, CHUNK)
    ypre = pre1(x.reshape(1, N, CHUNK), wrs[0], biases[0])
    h = post1(parts1, parts1, ypre, wls[0])
    for l in range(1, 4):
        parts = sc4(h.reshape(4 * N, CHUNK), srcoff4, dst_r, zeros_rows
                    ).reshape(NSC, 4, NP, CHUNK)
        ypre = pre4(h, wrs[l], biases[l])
        h = post4(parts, parts1, ypre, wls[l])

    hp = params['heads']
    w1 = jnp.concatenate([hp[n]['W1'] for n in _HEAD_NAMES], axis=1)
    b1 = jnp.concatenate([hp[n]['b1'] for n in _HEAD_NAMES]).reshape(1, 1280)
    w2 = jnp.zeros((1280, 128), jnp.float32)
    b2 = jnp.zeros((128,), jnp.float32)
    for k, n in enumerate(_HEAD_NAMES):
        w2 = w2.at[k * 256:(k + 1) * 256, k].set(hp[n]['W2'][:, 0])
        b2 = b2.at[k].set(hp[n]['b2'][0])
    res = _heads_tc(h, w1, b1, w2, b2.reshape(1, 128))
    return {n: res[:, k:k + 1] for k, n in enumerate(_HEAD_NAMES)}


# speedup vs baseline: 1.1577x; 1.1577x over previous
import functools

import jax
import jax.numpy as jnp
from jax import lax
from jax.experimental import pallas as pl
from jax.experimental.pallas import tpu as pltpu
from jax.experimental.pallas import tpu_sc as plsc

N = 10000
E = 320000
CHUNK = 128
NSC = 2
NTILES = 16
NW = NSC * NTILES
EPT = E // NW
B = 80
NB = EPT // B
NSEG = 5
NBS = NB // NSEG
NP = 10240
ROWS_PT = NP // NTILES
EPS = 1e-5
_HEAD_NAMES = ['port_surge', 'rail_congestion', 'terminal_utilization',
               'drayage_delay', 'chokepoint_risk']

@functools.cache
def _mesh():
    return plsc.VectorSubcoreMesh(core_axis_name="c", subcore_axis_name="s",
                                  num_cores=NSC, num_subcores=NTILES)


def _make_sc_agg(nc, with_deg):
    npass = nc + (1 if with_deg else 0)
    out_type = jax.ShapeDtypeStruct((NSC * npass * NP, CHUNK), jnp.float32)
    scratch = [
        pltpu.VMEM_SHARED((NP, CHUNK), jnp.float32),
        pltpu.VMEM((NBS, B), jnp.int32),
        pltpu.VMEM((NBS, B), jnp.int32),
        pltpu.VMEM((NBS, B), jnp.int32),
        pltpu.VMEM((NBS, B), jnp.int32),
        pltpu.VMEM((B, CHUNK), jnp.float32),
        pltpu.VMEM((B, CHUNK), jnp.float32),
        pltpu.SemaphoreType.DMA,
        pltpu.SemaphoreType.DMA,
        pltpu.SemaphoreType.DMA,
        pltpu.SemaphoreType.DMA,
        pltpu.SemaphoreType.DMA,
    ]
    if with_deg:
        scratch.append(pltpu.VMEM((B, CHUNK), jnp.float32))

    def body(h_hbm, srcoff_hbm, dst_hbm, zeros_hbm, parts_hbm, acc,
             src_v0, src_v1, dst_v0, dst_v1, msgs0, msgs1,
             sem0, sem1, ssem0, ssem1, isem, *rest):
        ones_v = rest[0] if with_deg else None
        c = lax.axis_index("c")
        s = lax.axis_index("s")
        wid = c * NTILES + s
        row0 = s * ROWS_PT

        if with_deg:
            ov = jnp.ones((16,), jnp.float32)

            def fill_ones(r, carry):
                for k in range(CHUNK // 16):
                    ones_v[r, k * 16:(k + 1) * 16] = ov
                return carry
            lax.fori_loop(0, B, fill_ones, 0)

        for cc in range(npass):
            pltpu.sync_copy(zeros_hbm, acc.at[pl.ds(row0, ROWS_PT)])
            plsc.subcore_barrier()

            if cc < nc:
                sbufs = (src_v0, src_v1)
                dbufs = (dst_v0, dst_v1)

                def iload(seg, issue):
                    sv, dv = sbufs[seg % 2], dbufs[seg % 2]
                    cps = [(dst_hbm.at[wid * NSEG + seg], dv),
                           (srcoff_hbm.at[(cc * NW + wid) * NSEG + seg], sv)]
                    for sref, dref in cps:
                        if issue:
                            pltpu.async_copy(sref, dref, isem)
                        else:
                            pltpu.make_async_copy(sref, dref, isem).wait()

                iload(0, True)
                for seg in range(NSEG):
                    src_v, dst_v = sbufs[seg % 2], dbufs[seg % 2]
                    iload(seg, False)
                    if seg + 1 < NSEG:
                        iload(seg + 1, True)

                    def gat(j, buf, sem):
                        pltpu.async_copy(h_hbm.at[src_v.at[j]], buf, sem)

                    def gwait(j, buf, sem):
                        pltpu.make_async_copy(h_hbm.at[src_v.at[j]], buf, sem
                                              ).wait()

                    def sstart(j, buf, sem):
                        pltpu.async_copy(buf, acc.at[dst_v.at[j]], sem,
                                         add=True)

                    def swait(j, buf, sem):
                        pltpu.make_async_copy(buf, acc.at[dst_v.at[j]], sem
                                              ).wait()

                    gat(0, msgs0, sem0)
                    gwait(0, msgs0, sem0)
                    sstart(0, msgs0, ssem0)
                    gat(1, msgs1, sem1)

                    def bstep(k, carry):
                        j = 2 * k + 1
                        gwait(j, msgs1, sem1)
                        sstart(j, msgs1, ssem1)
                        swait(j - 1, msgs0, ssem0)
                        gat(j + 1, msgs0, sem0)
                        gwait(j + 1, msgs0, sem0)
                        sstart(j + 1, msgs0, ssem0)
                        swait(j, msgs1, ssem1)
                        gat(j + 2, msgs1, sem1)
                        return carry
                    lax.fori_loop(0, (NBS - 3) // 2, bstep, 0)
                    j = NBS - 2
                    gwait(j, msgs1, sem1)
                    sstart(j, msgs1, ssem1)
                    swait(j - 1, msgs0, ssem0)
                    gat(j + 1, msgs0, sem0)
                    gwait(j + 1, msgs0, sem0)
                    sstart(j + 1, msgs0, ssem0)
                    swait(j, msgs1, ssem1)
                    swait(j + 1, msgs0, ssem0)
            else:
                for seg in range(NSEG):
                    pltpu.sync_copy(dst_hbm.at[wid * NSEG + seg], dst_v0)

                    def bstep(j, carry):
                        pltpu.sync_copy(ones_v, acc.at[dst_v0.at[j]], add=True)
                        return carry
                    lax.fori_loop(0, NBS, bstep, 0)

            plsc.subcore_barrier()
            pltpu.sync_copy(acc.at[pl.ds(row0, ROWS_PT)],
                            parts_hbm.at[pl.ds((c * npass + cc) * NP + row0, ROWS_PT)])
            if cc + 1 < npass:
                plsc.subcore_barrier()

    return pl.kernel(body, out_type=out_type, mesh=_mesh(),
                     scratch_types=tuple(scratch))


_R = 2000


def _pre_tc_body(nc, h_ref, wr_ref, b_ref, out_ref):
    h = jnp.concatenate([h_ref[i] for i in range(nc)], axis=-1)
    out_ref[...] = (jnp.dot(h, wr_ref[...], preferred_element_type=jnp.float32)
                    + b_ref[...])


def _make_pre_tc(nc):
    return pl.pallas_call(
        functools.partial(_pre_tc_body, nc),
        grid=(N // _R,),
        in_specs=[
            pl.BlockSpec((nc, _R, CHUNK), lambda i: (0, i, 0)),
            pl.BlockSpec((nc * CHUNK, 512), lambda i: (0, 0)),
            pl.BlockSpec((1, 512), lambda i: (0, 0)),
        ],
        out_specs=pl.BlockSpec((_R, 512), lambda i: (i, 0)),
        out_shape=jax.ShapeDtypeStruct((N, 512), jnp.float32),
    )


def _post_tc_body(nc, parts_ref, degp_ref, ypre_ref, wl_ref, out_ref):
    p = parts_ref[...]
    aggc = p[0] + p[1]
    agg = jnp.concatenate([aggc[i] for i in range(nc)], axis=-1)
    dp = degp_ref[...]
    deg = jnp.max(dp[0, 0] + dp[1, 0], axis=-1, keepdims=True)
    inv = 1.0 / jnp.maximum(deg, 1.0)
    y = jnp.dot(agg * inv, wl_ref[...], preferred_element_type=jnp.float32)
    y = jnp.maximum(y + ypre_ref[...], 0.0)
    for k in range(4):
        out_ref[k] = y[:, k * 128:(k + 1) * 128]


def _make_post_tc(nc):
    return pl.pallas_call(
        functools.partial(_post_tc_body, nc),
        grid=(N // _R,),
        in_specs=[
            pl.BlockSpec((NSC, nc, _R, CHUNK), lambda i: (0, 0, i, 0)),
            pl.BlockSpec((NSC, 1, _R, CHUNK), lambda i: (0, 1, i, 0)),
            pl.BlockSpec((_R, 512), lambda i: (i, 0)),
            pl.BlockSpec((nc * CHUNK, 512), lambda i: (0, 0)),
        ],
        out_specs=pl.BlockSpec((4, _R, CHUNK), lambda i: (0, i, 0)),
        out_shape=jax.ShapeDtypeStruct((4, N, CHUNK), jnp.float32),
    )


def _heads_tc_body(h_ref, w1_ref, b1_ref, w2_ref, b2_ref, out_ref):
    h = jnp.concatenate([h_ref[i] for i in range(4)], axis=-1)
    z = jnp.dot(h, w1_ref[...], preferred_element_type=jnp.float32)
    z = jnp.maximum(z + b1_ref[...], 0.0)
    o = jnp.dot(z, w2_ref[...], preferred_element_type=jnp.float32)
    out_ref[...] = jax.nn.sigmoid(o + b2_ref[...])


_heads_tc = pl.pallas_call(
    _heads_tc_body,
    grid=(N // _R,),
    in_specs=[
        pl.BlockSpec((4, _R, CHUNK), lambda i: (0, i, 0)),
        pl.BlockSpec((512, 1280), lambda i: (0, 0)),
        pl.BlockSpec((1, 1280), lambda i: (0, 0)),
        pl.BlockSpec((1280, 128), lambda i: (0, 0)),
        pl.BlockSpec((1, 128), lambda i: (0, 0)),
    ],
    out_specs=pl.BlockSpec((_R, 128), lambda i: (i, 0)),
    out_shape=jax.ShapeDtypeStruct((N, 128), jnp.float32),
)


def kernel(x, edge_index, params):
    src = edge_index[0]
    dst = edge_index[1]
    dst_r = dst.reshape(NW * NSEG, NBS, B)
    src_r = src.reshape(NW * NSEG, NBS, B)
    srcoff1 = src_r
    srcoff4 = (src_r[None] + (jnp.arange(4, dtype=jnp.int32) * N)[:, None, None, None]
               ).reshape(4 * NW * NSEG, NBS, B)
    zeros_rows = jnp.zeros((ROWS_PT, CHUNK), jnp.float32)

    wls, wrs, biases = [], [], []
    for lp in params['layers']:
        g = lp['gamma'] / jnp.sqrt(lp['rv'] + EPS)
        wls.append(lp['Wl'] * g[None, :])
        wrs.append(lp['Wr'] * g[None, :])
        biases.append(((lp['bl'] - lp['rm']) * g + lp['beta']).reshape(1, 512))

    sc1 = _make_sc_agg(1, True)
    sc4 = _make_sc_agg(4, False)
    pre1, post1 = _make_pre_tc(1), _make_post_tc(1)
    pre4, post4 = _make_pre_tc(4), _make_post_tc(4)

    parts1 = sc1(x, srcoff1, dst_r, zeros_rows).reshape(NSC, 2, NP, CHUNK)
    ypre = pre1(x.reshape(1, N, CHUNK), wrs[0], biases[0])
    h = post1(parts1, parts1, ypre, wls[0])
    for l in range(1, 4):
        parts = sc4(h.reshape(4 * N, CHUNK), srcoff4, dst_r, zeros_rows
                    ).reshape(NSC, 4, NP, CHUNK)
        ypre = pre4(h, wrs[l], biases[l])
        h = post4(parts, parts1, ypre, wls[l])

    hp = params['heads']
    w1 = jnp.concatenate([hp[n]['W1'] for n in _HEAD_NAMES], axis=1)
    b1 = jnp.concatenate([hp[n]['b1'] for n in _HEAD_NAMES]).reshape(1, 1280)
    w2 = jnp.zeros((1280, 128), jnp.float32)
    b2 = jnp.zeros((128,), jnp.float32)
    for k, n in enumerate(_HEAD_NAMES):
        w2 = w2.at[k * 256:(k + 1) * 256, k].set(hp[n]['W2'][:, 0])
        b2 = b2.at[k].set(hp[n]['b2'][0])
    res = _heads_tc(h, w1, b1, w2, b2.reshape(1, 128))
    return {n: res[:, k:k + 1] for k, n in enumerate(_HEAD_NAMES)}


# speedup vs baseline: 1.1628x; 1.0044x over previous
import functools

import jax
import jax.numpy as jnp
from jax import lax
from jax.experimental import pallas as pl
from jax.experimental.pallas import tpu as pltpu
from jax.experimental.pallas import tpu_sc as plsc

N = 10000
E = 320000
CHUNK = 128
NSC = 2
NTILES = 16
NW = NSC * NTILES
EPT = E // NW
B = 80
NB = EPT // B
NSEG = 5
NBS = NB // NSEG
NP = 10240
ROWS_PT = NP // NTILES
EPS = 1e-5
_HEAD_NAMES = ['port_surge', 'rail_congestion', 'terminal_utilization',
               'drayage_delay', 'chokepoint_risk']

@functools.cache
def _mesh():
    return plsc.VectorSubcoreMesh(core_axis_name="c", subcore_axis_name="s",
                                  num_cores=NSC, num_subcores=NTILES)


def _make_sc_agg(nc, with_deg):
    npass = nc + (1 if with_deg else 0)
    out_type = jax.ShapeDtypeStruct((NSC * npass * NP, CHUNK), jnp.float32)
    scratch = [
        pltpu.VMEM_SHARED((NP, CHUNK), jnp.float32),
        pltpu.VMEM((NBS, B), jnp.int32),
        pltpu.VMEM((NBS, B), jnp.int32),
        pltpu.VMEM((NBS, B), jnp.int32),
        pltpu.VMEM((NBS, B), jnp.int32),
        pltpu.VMEM((B, CHUNK), jnp.float32),
        pltpu.VMEM((B, CHUNK), jnp.float32),
        pltpu.SemaphoreType.DMA,
        pltpu.SemaphoreType.DMA,
        pltpu.SemaphoreType.DMA,
        pltpu.SemaphoreType.DMA,
        pltpu.SemaphoreType.DMA,
    ]
    if with_deg:
        scratch.append(pltpu.VMEM((B, CHUNK), jnp.float32))

    def body(h_hbm, srcoff_hbm, dst_hbm, zeros_hbm, parts_hbm, acc,
             src_v0, src_v1, dst_v0, dst_v1, msgs0, msgs1,
             sem0, sem1, ssem0, ssem1, isem, *rest):
        ones_v = rest[0] if with_deg else None
        c = lax.axis_index("c")
        s = lax.axis_index("s")
        wid = c * NTILES + s
        row0 = s * ROWS_PT

        if with_deg:
            ov = jnp.ones((16,), jnp.float32)

            def fill_ones(r, carry):
                for k in range(CHUNK // 16):
                    ones_v[r, k * 16:(k + 1) * 16] = ov
                return carry
            lax.fori_loop(0, B, fill_ones, 0)

        for cc in range(npass):
            pltpu.sync_copy(zeros_hbm, acc.at[pl.ds(row0, ROWS_PT)])
            plsc.subcore_barrier()

            if cc < nc:
                sbufs = (src_v0, src_v1)
                dbufs = (dst_v0, dst_v1)

                def iload(seg, issue):
                    sv, dv = sbufs[seg % 2], dbufs[seg % 2]
                    cps = [(dst_hbm.at[wid * NSEG + seg], dv),
                           (srcoff_hbm.at[(cc * NW + wid) * NSEG + seg], sv)]
                    for sref, dref in cps:
                        if issue:
                            pltpu.async_copy(sref, dref, isem)
                        else:
                            pltpu.make_async_copy(sref, dref, isem).wait()

                iload(0, True)
                for seg in range(NSEG):
                    src_v, dst_v = sbufs[seg % 2], dbufs[seg % 2]
                    iload(seg, False)
                    if seg + 1 < NSEG:
                        iload(seg + 1, True)

                    def gat(j, buf, sem):
                        pltpu.async_copy(h_hbm.at[src_v.at[j]], buf, sem)

                    def gwait(j, buf, sem):
                        pltpu.make_async_copy(h_hbm.at[src_v.at[j]], buf, sem
                                              ).wait()

                    def sstart(j, buf, sem):
                        pltpu.async_copy(buf, acc.at[dst_v.at[j]], sem,
                                         add=True)

                    def swait(j, buf, sem):
                        pltpu.make_async_copy(buf, acc.at[dst_v.at[j]], sem
                                              ).wait()

                    gat(0, msgs0, sem0)
                    gwait(0, msgs0, sem0)
                    sstart(0, msgs0, ssem0)
                    gat(1, msgs1, sem1)

                    def bstep(k, carry):
                        j = 2 * k + 1
                        gwait(j, msgs1, sem1)
                        sstart(j, msgs1, ssem1)
                        swait(j - 1, msgs0, ssem0)
                        gat(j + 1, msgs0, sem0)
                        gwait(j + 1, msgs0, sem0)
                        sstart(j + 1, msgs0, ssem0)
                        swait(j, msgs1, ssem1)
                        gat(j + 2, msgs1, sem1)
                        return carry
                    lax.fori_loop(0, (NBS - 3) // 2, bstep, 0)
                    j = NBS - 2
                    gwait(j, msgs1, sem1)
                    sstart(j, msgs1, ssem1)
                    swait(j - 1, msgs0, ssem0)
                    gat(j + 1, msgs0, sem0)
                    gwait(j + 1, msgs0, sem0)
                    sstart(j + 1, msgs0, ssem0)
                    swait(j, msgs1, ssem1)
                    swait(j + 1, msgs0, ssem0)
            else:
                for seg in range(NSEG):
                    pltpu.sync_copy(dst_hbm.at[wid * NSEG + seg], dst_v0)

                    def bstep(j, carry):
                        pltpu.sync_copy(ones_v, acc.at[dst_v0.at[j]], add=True)
                        return carry
                    lax.fori_loop(0, NBS, bstep, 0)

            plsc.subcore_barrier()
            pltpu.sync_copy(acc.at[pl.ds(row0, ROWS_PT)],
                            parts_hbm.at[pl.ds((c * npass + cc) * NP + row0, ROWS_PT)])
            if cc + 1 < npass:
                plsc.subcore_barrier()

    return pl.kernel(body, out_type=out_type, mesh=_mesh(),
                     scratch_types=tuple(scratch))


_R = 2000


def _pre_tc_body(nc, h_ref, wr_ref, b_ref, out_ref):
    h = jnp.concatenate([h_ref[i] for i in range(nc)], axis=-1)
    out_ref[...] = (jnp.dot(h, wr_ref[...], preferred_element_type=jnp.float32)
                    + b_ref[...])


def _make_pre_tc(nc):
    return pl.pallas_call(
        functools.partial(_pre_tc_body, nc),
        grid=(N // _R,),
        in_specs=[
            pl.BlockSpec((nc, _R, CHUNK), lambda i: (0, i, 0)),
            pl.BlockSpec((nc * CHUNK, 512), lambda i: (0, 0)),
            pl.BlockSpec((1, 512), lambda i: (0, 0)),
        ],
        out_specs=pl.BlockSpec((_R, 512), lambda i: (i, 0)),
        out_shape=jax.ShapeDtypeStruct((N, 512), jnp.float32),
    )


def _sage_y(nc, parts_ref, degp_ref, ypre_ref, wl_ref):
    p = parts_ref[...]
    aggc = p[0] + p[1]
    agg = jnp.concatenate([aggc[i] for i in range(nc)], axis=-1)
    dp = degp_ref[...]
    deg = jnp.max(dp[0, 0] + dp[1, 0], axis=-1, keepdims=True)
    inv = 1.0 / jnp.maximum(deg, 1.0)
    y = jnp.dot(agg * inv, wl_ref[...], preferred_element_type=jnp.float32)
    return jnp.maximum(y + ypre_ref[...], 0.0)


def _fused_tc_body(nc, parts_ref, degp_ref, ypre_ref, wl_ref, wrn_ref,
                   bn_ref, out_ref, ypre_out_ref):
    y = _sage_y(nc, parts_ref, degp_ref, ypre_ref, wl_ref)
    for k in range(4):
        out_ref[k] = y[:, k * 128:(k + 1) * 128]
    ypre_out_ref[...] = (jnp.dot(y, wrn_ref[...],
                                 preferred_element_type=jnp.float32)
                         + bn_ref[...])


def _make_fused_tc(nc):
    return pl.pallas_call(
        functools.partial(_fused_tc_body, nc),
        grid=(N // _R,),
        in_specs=[
            pl.BlockSpec((NSC, nc, _R, CHUNK), lambda i: (0, 0, i, 0)),
            pl.BlockSpec((NSC, 1, _R, CHUNK), lambda i: (0, 1, i, 0)),
            pl.BlockSpec((_R, 512), lambda i: (i, 0)),
            pl.BlockSpec((nc * CHUNK, 512), lambda i: (0, 0)),
            pl.BlockSpec((512, 512), lambda i: (0, 0)),
            pl.BlockSpec((1, 512), lambda i: (0, 0)),
        ],
        out_specs=[pl.BlockSpec((4, _R, CHUNK), lambda i: (0, i, 0)),
                   pl.BlockSpec((_R, 512), lambda i: (i, 0))],
        out_shape=[jax.ShapeDtypeStruct((4, N, CHUNK), jnp.float32),
                   jax.ShapeDtypeStruct((N, 512), jnp.float32)],
    )


def _last_tc_body(parts_ref, degp_ref, ypre_ref, wl_ref, w1_ref, b1_ref,
                  w2_ref, b2_ref, out_ref):
    y = _sage_y(4, parts_ref, degp_ref, ypre_ref, wl_ref)
    z = jnp.dot(y, w1_ref[...], preferred_element_type=jnp.float32)
    z = jnp.maximum(z + b1_ref[...], 0.0)
    o = jnp.dot(z, w2_ref[...], preferred_element_type=jnp.float32)
    out_ref[...] = jax.nn.sigmoid(o + b2_ref[...])


_last_tc = pl.pallas_call(
    _last_tc_body,
    grid=(N // _R,),
    in_specs=[
        pl.BlockSpec((NSC, 4, _R, CHUNK), lambda i: (0, 0, i, 0)),
        pl.BlockSpec((NSC, 1, _R, CHUNK), lambda i: (0, 1, i, 0)),
        pl.BlockSpec((_R, 512), lambda i: (i, 0)),
        pl.BlockSpec((512, 512), lambda i: (0, 0)),
        pl.BlockSpec((512, 1280), lambda i: (0, 0)),
        pl.BlockSpec((1, 1280), lambda i: (0, 0)),
        pl.BlockSpec((1280, 128), lambda i: (0, 0)),
        pl.BlockSpec((1, 128), lambda i: (0, 0)),
    ],
    out_specs=pl.BlockSpec((_R, 128), lambda i: (i, 0)),
    out_shape=jax.ShapeDtypeStruct((N, 128), jnp.float32),
)


def kernel(x, edge_index, params):
    src = edge_index[0]
    dst = edge_index[1]
    dst_r = dst.reshape(NW * NSEG, NBS, B)
    src_r = src.reshape(NW * NSEG, NBS, B)
    srcoff1 = src_r
    srcoff4 = (src_r[None] + (jnp.arange(4, dtype=jnp.int32) * N)[:, None, None, None]
               ).reshape(4 * NW * NSEG, NBS, B)
    zeros_rows = jnp.zeros((ROWS_PT, CHUNK), jnp.float32)

    wls, wrs, biases = [], [], []
    for lp in params['layers']:
        g = lp['gamma'] / jnp.sqrt(lp['rv'] + EPS)
        wls.append(lp['Wl'] * g[None, :])
        wrs.append(lp['Wr'] * g[None, :])
        biases.append(((lp['bl'] - lp['rm']) * g + lp['beta']).reshape(1, 512))

    hp = params['heads']
    w1 = jnp.concatenate([hp[n]['W1'] for n in _HEAD_NAMES], axis=1)
    b1 = jnp.concatenate([hp[n]['b1'] for n in _HEAD_NAMES]).reshape(1, 1280)
    w2 = jnp.zeros((1280, 128), jnp.float32)
    b2 = jnp.zeros((128,), jnp.float32)
    for k, n in enumerate(_HEAD_NAMES):
        w2 = w2.at[k * 256:(k + 1) * 256, k].set(hp[n]['W2'][:, 0])
        b2 = b2.at[k].set(hp[n]['b2'][0])

    sc1 = _make_sc_agg(1, True)
    sc4 = _make_sc_agg(4, False)
    pre1 = _make_pre_tc(1)
    f1, f4 = _make_fused_tc(1), _make_fused_tc(4)

    parts1 = sc1(x, srcoff1, dst_r, zeros_rows).reshape(NSC, 2, NP, CHUNK)
    ypre = pre1(x.reshape(1, N, CHUNK), wrs[0], biases[0])
    h, ypre = f1(parts1, parts1, ypre, wls[0], wrs[1], biases[1])
    for l in (1, 2):
        parts = sc4(h.reshape(4 * N, CHUNK), srcoff4, dst_r, zeros_rows
                    ).reshape(NSC, 4, NP, CHUNK)
        h, ypre = f4(parts, parts1, ypre, wls[l], wrs[l + 1], biases[l + 1])
    parts = sc4(h.reshape(4 * N, CHUNK), srcoff4, dst_r, zeros_rows
                ).reshape(NSC, 4, NP, CHUNK)
    res = _last_tc(parts, parts1, ypre, wls[3], w1, b1, w2, b2.reshape(1, 128))
    return {n: res[:, k:k + 1] for k, n in enumerate(_HEAD_NAMES)}
